# baseline (speedup 1.0000x reference)
import functools

import jax
import jax.numpy as jnp
from jax import lax
from jax.experimental import pallas as pl
from jax.experimental.pallas import tpu as pltpu
from jax.experimental.pallas import tpu_sc as plsc

_N = 16384
_NSC = 4096
_NTC = _N - _NSC
_BN = 2048
_R = 8
_NW = 32


def _tc_body(hs_ref, tv_ref, idx_ref, wd_ref, wu_ref, alias_ref, out_ref):
    del alias_ref
    bN = hs_ref.shape[0]
    rank = wd_ref.shape[0]
    down = lax.dot_general(
        hs_ref[...], wd_ref[...], (((1,), (1,)), ((), ())),
        preferred_element_type=jnp.float32)
    iota = lax.broadcasted_iota(jnp.int32, (bN, rank), 1)
    idx = idx_ref[...]
    tv = tv_ref[...]
    w = jnp.zeros((bN, rank), jnp.float32)
    for k in range(idx.shape[1]):
        w = jnp.where(iota == idx[:, k:k + 1], tv[:, k:k + 1], w)
    out_ref[...] = lax.dot_general(
        down * w, wu_ref[...], (((1,), (1,)), ((), ())),
        preferred_element_type=jnp.float32)


def _sc_body(hs_hbm, idx_hbm, tv_hbm, wcat_hbm, out_hbm,
             hs_buf, g_buf, out_buf, idx_buf, tv_buf, gl_buf, sem):
    wid = lax.axis_index("s") * 2 + lax.axis_index("c")
    t_per_w = _NSC // _NW
    base = _NTC + wid * t_per_w
    lane = lax.iota(jnp.int32, 16)

    def chunk(ch, carry):
        tok0 = base + ch * _R
        pltpu.sync_copy(hs_hbm.at[pl.ds(tok0, _R)], hs_buf)
        pltpu.sync_copy(idx_hbm.at[pl.ds(tok0 * 2, 2 * _R)], idx_buf)
        pltpu.sync_copy(tv_hbm.at[pl.ds(tok0 * 2, 2 * _R)], tv_buf)
        iv = idx_buf[...]
        gl_buf[pl.ds(0, 16)] = iv
        gl_buf[pl.ds(16, 16)] = iv + 64
        pltpu.async_copy(wcat_hbm.at[gl_buf], g_buf, sem).wait()
        tvv = tv_buf[...]
        for r in range(_R):
            def dot_step(c, acc, r=r):
                h = hs_buf[r, pl.ds(c * 16, 16)]
                return (acc[0] + h * g_buf[2 * r, pl.ds(c * 16, 16)],
                        acc[1] + h * g_buf[2 * r + 1, pl.ds(c * 16, 16)])
            z = jnp.zeros((16,), jnp.float32)
            a0v, a1v = lax.fori_loop(0, 64, dot_step, (z, z))
            v0 = jnp.sum(jnp.where(lane == 2 * r, tvv, 0.0))
            v1 = jnp.sum(jnp.where(lane == 2 * r + 1, tvv, 0.0))
            a0 = v0 * jnp.sum(a0v)
            a1 = v1 * jnp.sum(a1v)

            def up_step(c, _, r=r, a0=a0, a1=a1):
                out_buf[r, pl.ds(c * 16, 16)] = (
                    a0 * g_buf[16 + 2 * r, pl.ds(c * 16, 16)]
                    + a1 * g_buf[17 + 2 * r, pl.ds(c * 16, 16)])
                return 0
            lax.fori_loop(0, 64, up_step, 0)
        pltpu.sync_copy(out_buf, out_hbm.at[pl.ds(tok0, _R)])
        return carry

    lax.fori_loop(0, t_per_w // _R, chunk, 0)


def _sc_call(hs, idx_flat, tv_flat, wcat):
    mesh = plsc.VectorSubcoreMesh(core_axis_name="c", subcore_axis_name="s")
    f = functools.partial(
        pl.kernel,
        out_type=jax.ShapeDtypeStruct((_N, 1024), jnp.float32),
        mesh=mesh,
        scratch_types=[
            pltpu.VMEM((_R, 1024), jnp.float32),
            pltpu.VMEM((4 * _R, 1024), jnp.float32),
            pltpu.VMEM((_R, 1024), jnp.float32),
            pltpu.VMEM((2 * _R,), jnp.int32),
            pltpu.VMEM((2 * _R,), jnp.float32),
            pltpu.VMEM((4 * _R,), jnp.int32),
            pltpu.SemaphoreType.DMA,
        ],
        compiler_params=pltpu.CompilerParams(needs_layout_passes=False),
    )(_sc_body)
    return f(hs, idx_flat, tv_flat, wcat)


def kernel(hidden_states, top_k_values, top_k_indices, W_down, W_up):
    N, d_in = hidden_states.shape
    rank, _ = W_down.shape
    d_out, _ = W_up.shape
    top_k = top_k_values.shape[1]
    idx = top_k_indices.astype(jnp.int32)

    tv_eff = jnp.where(
        jnp.arange(top_k)[None, :] < top_k - 1,
        jnp.where(idx[:, :1] == idx[:, 1:], 0.0, top_k_values),
        top_k_values)
    wcat = jnp.concatenate([W_down, W_up.T], axis=0)
    out_sc = _sc_call(hidden_states, jnp.reshape(idx, (-1,)),
                      jnp.reshape(tv_eff, (-1,)), wcat)

    grid = (_NTC // _BN,)
    return pl.pallas_call(
        _tc_body,
        grid=grid,
        in_specs=[
            pl.BlockSpec((_BN, d_in), lambda i: (i, 0)),
            pl.BlockSpec((_BN, top_k), lambda i: (i, 0)),
            pl.BlockSpec((_BN, top_k), lambda i: (i, 0)),
            pl.BlockSpec((rank, d_in), lambda i: (0, 0)),
            pl.BlockSpec((d_out, rank), lambda i: (0, 0)),
            pl.BlockSpec(memory_space=pltpu.MemorySpace.HBM),
        ],
        out_specs=pl.BlockSpec((_BN, d_out), lambda i: (i, 0)),
        out_shape=jax.ShapeDtypeStruct((N, d_out), jnp.float32),
        input_output_aliases={5: 0},
        compiler_params=pltpu.CompilerParams(
            dimension_semantics=("arbitrary",),
        ),
    )(hidden_states, top_k_values, idx, W_down, W_up, out_sc)


# speedup vs baseline: 3.4619x; 3.4619x over previous
import jax
import jax.numpy as jnp
from jax.experimental import pallas as pl
from jax.experimental.pallas import tpu as pltpu


def _body(hs_ref, tv_ref, idx_ref, wd_ref, wu_ref, out_ref):
    bN = hs_ref.shape[0]
    rank = wd_ref.shape[0]
    down = jax.lax.dot_general(
        hs_ref[...].astype(jnp.bfloat16), wd_ref[...].astype(jnp.bfloat16),
        (((1,), (1,)), ((), ())),
        preferred_element_type=jnp.float32)
    iota = jax.lax.broadcasted_iota(jnp.int32, (bN, rank), 1)
    idx = idx_ref[...]
    tv = tv_ref[...]
    w = jnp.zeros((bN, rank), jnp.float32)
    top_k = idx.shape[1]
    for k in range(top_k):
        w = jnp.where(iota == idx[:, k:k + 1], tv[:, k:k + 1], w)
    out_ref[...] = jax.lax.dot_general(
        (down * w).astype(jnp.bfloat16), wu_ref[...].astype(jnp.bfloat16),
        (((1,), (1,)), ((), ())),
        preferred_element_type=jnp.float32)


def kernel(hidden_states, top_k_values, top_k_indices, W_down, W_up):
    N, d_in = hidden_states.shape
    rank, _ = W_down.shape
    d_out, _ = W_up.shape
    top_k = top_k_values.shape[1]
    bN = 2048
    grid = (N // bN,)
    return pl.pallas_call(
        _body,
        grid=grid,
        in_specs=[
            pl.BlockSpec((bN, d_in), lambda i: (i, 0)),
            pl.BlockSpec((bN, top_k), lambda i: (i, 0)),
            pl.BlockSpec((bN, top_k), lambda i: (i, 0)),
            pl.BlockSpec((rank, d_in), lambda i: (0, 0)),
            pl.BlockSpec((d_out, rank), lambda i: (0, 0)),
        ],
        out_specs=pl.BlockSpec((bN, d_out), lambda i: (i, 0)),
        out_shape=jax.ShapeDtypeStruct((N, d_out), jnp.float32),
        compiler_params=pltpu.CompilerParams(
            dimension_semantics=("arbitrary",),
        ),
    )(hidden_states, top_k_values, top_k_indices.astype(jnp.int32),
      W_down, W_up)
